# baseline (speedup 1.0000x reference)
import functools

import jax
import jax.numpy as jnp
from jax import lax
from jax.experimental import pallas as pl
from jax.experimental.pallas import tpu as pltpu
from jax.experimental.pallas import tpu_sc as plsc

_CTX = 200
_DIM = 64
_ROWS_PER_TILE = 8
_N_TITLE_TILES = _CTX // _ROWS_PER_TILE


def _gather_body(idx_hbm, cidx_hbm, emb_hbm, emb2_hbm, out_hbm, cat_hbm,
                 idx_v, rows_v, sem, *, num_cores):
    wid = lax.axis_index("s") * num_cores + lax.axis_index("c")

    @pl.when(wid < _N_TITLE_TILES)
    def _():
        base = wid * _ROWS_PER_TILE
        pltpu.sync_copy(idx_hbm.at[pl.ds(base, _ROWS_PER_TILE)], idx_v)
        pltpu.async_copy(emb_hbm.at[idx_v], rows_v, sem).wait()
        pltpu.sync_copy(rows_v, out_hbm.at[pl.ds(base, _ROWS_PER_TILE)])

    @pl.when(wid == _N_TITLE_TILES)
    def _():
        pltpu.sync_copy(cidx_hbm, idx_v)
        pltpu.async_copy(emb2_hbm.at[idx_v], rows_v, sem).wait()
        pltpu.sync_copy(rows_v, cat_hbm)


def _make_sc_gather():
    mesh = plsc.VectorSubcoreMesh(core_axis_name="c", subcore_axis_name="s")
    return pl.kernel(
        functools.partial(_gather_body, num_cores=mesh.num_cores),
        out_type=[
            jax.ShapeDtypeStruct((_CTX, _DIM), jnp.float32),
            jax.ShapeDtypeStruct((_ROWS_PER_TILE, _DIM), jnp.float32),
        ],
        mesh=mesh,
        compiler_params=pltpu.CompilerParams(use_tc_tiling_on_sc=False),
        scratch_types=[
            pltpu.VMEM((_ROWS_PER_TILE,), jnp.int32),
            pltpu.VMEM((_ROWS_PER_TILE, _DIM), jnp.float32),
            pltpu.SemaphoreType.DMA,
        ],
    )


def _mlp_body(x_ref, w1_ref, b1_ref, w2_ref, b2_ref, o_ref):
    h = jnp.dot(x_ref[...], w1_ref[...], preferred_element_type=jnp.float32)
    h = jnp.maximum(h + b1_ref[...], 0.0)
    o = jnp.dot(h, w2_ref[...], preferred_element_type=jnp.float32) + b2_ref[...]
    o_ref[...] = jax.nn.sigmoid(o)


def _mlp(x, W1, b1, W2, b2):
    return pl.pallas_call(
        _mlp_body,
        out_shape=jax.ShapeDtypeStruct((1, 1), jnp.float32),
    )(x, W1, b1, W2, b2)


def kernel(category, title, quantity, emb, emb2, W1, b1, W2, b2):
    title_idx = title.astype(jnp.int32)
    cat_idx = jnp.broadcast_to(category.astype(jnp.int32), (_ROWS_PER_TILE,))
    gathered, cat_rows = _make_sc_gather()(title_idx, cat_idx, emb, emb2)
    x = jnp.concatenate(
        [cat_rows[0].reshape(1, _DIM),
         gathered.reshape(1, _CTX * _DIM),
         quantity.reshape(1, 1)], axis=1)
    return _mlp(x, W1, b1.reshape(1, -1), W2, b2.reshape(1, 1))


# speedup vs baseline: 1.6884x; 1.6884x over previous
import functools

import jax
import jax.numpy as jnp
from jax import lax
from jax.experimental import pallas as pl
from jax.experimental.pallas import tpu as pltpu
from jax.experimental.pallas import tpu_sc as plsc

_CTX = 200
_DIM = 64
_ROWS_PER_TILE = 16
_CTX_PAD = 208
_N_TITLE_TILES = _CTX_PAD // _ROWS_PER_TILE


def _gather_body(idx_hbm, cidx_hbm, emb_hbm, emb2_hbm, out_hbm, cat_hbm,
                 idx_v, rows_v, sem, *, num_cores):
    wid = lax.axis_index("s") * num_cores + lax.axis_index("c")

    @pl.when(wid < _N_TITLE_TILES)
    def _():
        base = wid * _ROWS_PER_TILE
        pltpu.sync_copy(idx_hbm.at[pl.ds(base, _ROWS_PER_TILE)], idx_v)
        idx = idx_v[...]
        copies = [
            pltpu.async_copy(emb_hbm.at[pl.ds(idx[j], 1)],
                             rows_v.at[pl.ds(j, 1)], sem)
            for j in range(_ROWS_PER_TILE)
        ]
        for c in copies:
            c.wait()
        pltpu.sync_copy(rows_v, out_hbm.at[pl.ds(base, _ROWS_PER_TILE)])

    @pl.when(wid == _N_TITLE_TILES)
    def _():
        pltpu.sync_copy(cidx_hbm, idx_v)
        pltpu.async_copy(emb2_hbm.at[pl.ds(idx_v[...][0], 1)],
                         rows_v.at[pl.ds(0, 1)], sem).wait()
        pltpu.sync_copy(rows_v, cat_hbm)


def _make_sc_gather():
    mesh = plsc.VectorSubcoreMesh(core_axis_name="c", subcore_axis_name="s")
    return pl.kernel(
        functools.partial(_gather_body, num_cores=mesh.num_cores),
        out_type=[
            jax.ShapeDtypeStruct((_CTX_PAD, _DIM), jnp.float32),
            jax.ShapeDtypeStruct((_ROWS_PER_TILE, _DIM), jnp.float32),
        ],
        mesh=mesh,
        scratch_types=[
            pltpu.VMEM((_ROWS_PER_TILE,), jnp.int32),
            pltpu.VMEM((_ROWS_PER_TILE, _DIM), jnp.float32),
            pltpu.SemaphoreType.DMA,
        ],
    )


def _mlp_body(x_ref, w1_ref, b1_ref, w2_ref, b2_ref, o_ref):
    h = jnp.dot(x_ref[...], w1_ref[...], preferred_element_type=jnp.float32)
    h = jnp.maximum(h + b1_ref[...], 0.0)
    o = jnp.dot(h, w2_ref[...], preferred_element_type=jnp.float32) + b2_ref[...]
    o_ref[...] = jax.nn.sigmoid(o)


def _mlp(x, W1, b1, W2, b2):
    return pl.pallas_call(
        _mlp_body,
        out_shape=jax.ShapeDtypeStruct((1, 1), jnp.float32),
    )(x, W1, b1, W2, b2)


def kernel(category, title, quantity, emb, emb2, W1, b1, W2, b2):
    title_idx = jnp.concatenate(
        [title.astype(jnp.int32),
         jnp.zeros((_CTX_PAD - _CTX,), jnp.int32)])
    cat_idx = jnp.broadcast_to(category.astype(jnp.int32), (_ROWS_PER_TILE,))
    gathered, cat_rows = _make_sc_gather()(title_idx, cat_idx, emb, emb2)
    x = jnp.concatenate(
        [cat_rows[0].reshape(1, _DIM),
         gathered[:_CTX].reshape(1, _CTX * _DIM),
         quantity.reshape(1, 1)], axis=1)
    return _mlp(x, W1, b1.reshape(1, -1), W2, b2.reshape(1, 1))


# speedup vs baseline: 11.9441x; 7.0741x over previous
import functools

import jax
import jax.numpy as jnp
from jax import lax
from jax.experimental import pallas as pl
from jax.experimental.pallas import tpu as pltpu
from jax.experimental.pallas import tpu_sc as plsc

_CTX = 200
_DIM = 64
_HID = 128
_LANES = 16
_TOK_PER_TILE = 8
_N_TITLE_TILES = _CTX // _TOK_PER_TILE
_NROW_CHUNKS = _DIM // _LANES


def _gather_body(idx_hbm, embT_hbm, x_hbm, idx_v, win_v, xrow_v, sem,
                 *, num_cores):
    wid = lax.axis_index("s") * num_cores + lax.axis_index("c")

    @pl.when(wid < _N_TITLE_TILES)
    def _():
        base = wid * _TOK_PER_TILE
        pltpu.sync_copy(idx_hbm.at[pl.ds(wid * _LANES, _LANES)], idx_v)
        idx = idx_v[...]
        copies = []
        for j in range(_TOK_PER_TILE):
            colb = pl.multiple_of((idx[j] // 128) * 128, 128)
            copies.append(
                pltpu.async_copy(embT_hbm.at[:, pl.ds(colb, 128)],
                                 win_v.at[j], sem))
        rows0 = lax.iota(jnp.int32, _LANES)
        for j in range(_TOK_PER_TILE):
            copies[j].wait()
            off = jnp.full((_LANES,), idx[j] % 128, jnp.int32)
            for b in range(_NROW_CHUNKS):
                vals = plsc.load_gather(win_v.at[j], [rows0 + b * _LANES, off])
                xrow_v[j, pl.ds(b * _LANES, _LANES)] = vals
        pltpu.sync_copy(xrow_v, x_hbm.at[pl.ds(base, _TOK_PER_TILE)])


def _make_sc_gather():
    mesh = plsc.VectorSubcoreMesh(core_axis_name="c", subcore_axis_name="s")
    return pl.kernel(
        functools.partial(_gather_body, num_cores=mesh.num_cores),
        out_type=jax.ShapeDtypeStruct((_CTX, _DIM), jnp.float32),
        mesh=mesh,
        compiler_params=pltpu.CompilerParams(disable_bounds_checks=True,
                                             needs_layout_passes=False),
        scratch_types=[
            pltpu.VMEM((_LANES,), jnp.int32),
            pltpu.VMEM((_TOK_PER_TILE, _DIM, 128), jnp.float32),
            pltpu.VMEM((_TOK_PER_TILE, _DIM), jnp.float32),
            pltpu.SemaphoreType.DMA,
        ],
    )


def _mlp_body(x_ref, emb2T_ref, cat_ref, q_ref, w1_ref, b1_ref, w2_ref,
              b2_ref, o_ref):
    ncat = emb2T_ref.shape[1]
    lane = lax.broadcasted_iota(jnp.int32, (_DIM, ncat), 1)
    col = jnp.where(lane == cat_ref[0], emb2T_ref[...], 0.0)
    cat_row = jnp.sum(col, axis=1, keepdims=True)
    h0 = jnp.sum(cat_row * w1_ref[0:_DIM, :], axis=0, keepdims=True)

    def step(r, h):
        xr = x_ref[pl.ds(r, 1), :]
        wr = w1_ref[pl.ds(_DIM + r * _DIM, _DIM), :]
        return h + jnp.dot(xr, wr, preferred_element_type=jnp.float32)

    h = lax.fori_loop(0, _CTX, step, h0)
    h = h + q_ref[...] * w1_ref[_CTX * _DIM + _DIM:_CTX * _DIM + _DIM + 1, :]
    h = jnp.maximum(h + b1_ref[...], 0.0)
    o = jnp.dot(h, w2_ref[...], preferred_element_type=jnp.float32)
    o_ref[...] = jax.nn.sigmoid(o + b2_ref[...])


def _mlp(x, emb2T, cat, q, W1, b1, W2, b2):
    vmem = pl.BlockSpec(memory_space=pltpu.MemorySpace.VMEM)
    return pl.pallas_call(
        _mlp_body,
        in_specs=[vmem, vmem,
                  pl.BlockSpec(memory_space=pltpu.MemorySpace.SMEM),
                  vmem, vmem, vmem, vmem, vmem],
        out_shape=jax.ShapeDtypeStruct((1, 1), jnp.float32),
    )(x, emb2T, cat, q, W1, b1, W2, b2)


def kernel(category, title, quantity, emb, emb2, W1, b1, W2, b2):
    idx = jnp.pad(title.astype(jnp.int32).reshape(_N_TITLE_TILES, 8),
                  ((0, 0), (0, 8))).reshape(-1)
    x = _make_sc_gather()(idx, emb.T)
    return _mlp(x, emb2.T, category.astype(jnp.int32),
                quantity.reshape(1, 1), W1, b1.reshape(1, _HID),
                W2, b2.reshape(1, 1))


# speedup vs baseline: 22.4737x; 1.8816x over previous
import functools

import jax
import jax.numpy as jnp
from jax import lax
from jax.experimental import pallas as pl
from jax.experimental.pallas import tpu as pltpu
from jax.experimental.pallas import tpu_sc as plsc

_CTX = 200
_DIM = 64
_HID = 128
_LANES = 16
_TOK_PER_TILE = 8
_N_TITLE_TILES = _CTX // _TOK_PER_TILE
_NROW_CHUNKS = _DIM // _LANES


def _gather_body(idx_hbm, embT_hbm, x_hbm, idx_v, win_v, xflat_v, sem,
                 *, num_cores):
    wid = lax.axis_index("s") * num_cores + lax.axis_index("c")

    @pl.when(wid < _N_TITLE_TILES)
    def _():
        base = wid * (_TOK_PER_TILE * _DIM)
        pltpu.sync_copy(idx_hbm.at[pl.ds(wid * _LANES, _LANES)], idx_v)
        idx = idx_v[...]
        copies = []
        for j in range(_TOK_PER_TILE):
            colb = pl.multiple_of((idx[j] // 128) * 128, 128)
            copies.append(
                pltpu.async_copy(embT_hbm.at[:, pl.ds(colb, 128)],
                                 win_v.at[j], sem))
        rows0 = lax.iota(jnp.int32, _LANES)
        for j in range(_TOK_PER_TILE):
            copies[j].wait()
            off = jnp.full((_LANES,), idx[j] % 128, jnp.int32)
            for b in range(_NROW_CHUNKS):
                vals = plsc.load_gather(win_v.at[j], [rows0 + b * _LANES, off])
                xflat_v[0, pl.ds(j * _DIM + b * _LANES, _LANES)] = vals
        pltpu.sync_copy(
            xflat_v,
            x_hbm.at[:, pl.ds(pl.multiple_of(base, 128), _TOK_PER_TILE * _DIM)])


def _make_sc_gather():
    mesh = plsc.VectorSubcoreMesh(core_axis_name="c", subcore_axis_name="s")
    return pl.kernel(
        functools.partial(_gather_body, num_cores=mesh.num_cores),
        out_type=jax.ShapeDtypeStruct((1, _CTX * _DIM), jnp.float32),
        mesh=mesh,
        compiler_params=pltpu.CompilerParams(disable_bounds_checks=True,
                                             needs_layout_passes=False),
        scratch_types=[
            pltpu.VMEM((_LANES,), jnp.int32),
            pltpu.VMEM((_TOK_PER_TILE, _DIM, 128), jnp.float32),
            pltpu.VMEM((1, _TOK_PER_TILE * _DIM), jnp.float32),
            pltpu.SemaphoreType.DMA,
        ],
    )


def _mlp_body(x_ref, emb2T_ref, cat_ref, q_ref, w1_ref, b1_ref, w2_ref,
              b2_ref, o_ref):
    ncat = emb2T_ref.shape[1]
    lane = lax.broadcasted_iota(jnp.int32, (_DIM, ncat), 1)
    col = jnp.where(lane == cat_ref[0], emb2T_ref[...], 0.0)
    cat_row = jnp.sum(col, axis=1, keepdims=True)
    h0 = jnp.sum(cat_row * w1_ref[0:_DIM, :], axis=0, keepdims=True)

    h = h0 + jnp.dot(x_ref[...], w1_ref[_DIM:_DIM + _CTX * _DIM, :],
                     preferred_element_type=jnp.float32)
    h = h + q_ref[...] * w1_ref[_CTX * _DIM + _DIM:_CTX * _DIM + _DIM + 1, :]
    h = jnp.maximum(h + b1_ref[...], 0.0)
    o = jnp.dot(h, w2_ref[...], preferred_element_type=jnp.float32)
    o_ref[...] = jax.nn.sigmoid(o + b2_ref[...])


def _mlp(x, emb2T, cat, q, W1, b1, W2, b2):
    vmem = pl.BlockSpec(memory_space=pltpu.MemorySpace.VMEM)
    return pl.pallas_call(
        _mlp_body,
        in_specs=[vmem, vmem,
                  pl.BlockSpec(memory_space=pltpu.MemorySpace.SMEM),
                  vmem, vmem, vmem, vmem, vmem],
        out_shape=jax.ShapeDtypeStruct((1, 1), jnp.float32),
    )(x, emb2T, cat, q, W1, b1, W2, b2)


def kernel(category, title, quantity, emb, emb2, W1, b1, W2, b2):
    idx = jnp.pad(title.astype(jnp.int32).reshape(_N_TITLE_TILES, 8),
                  ((0, 0), (0, 8))).reshape(-1)
    x = _make_sc_gather()(idx, emb.T)
    return _mlp(x, emb2.T, category.astype(jnp.int32),
                quantity.reshape(1, 1), W1, b1.reshape(1, _HID),
                W2, b2.reshape(1, 1))
